# baseline (speedup 1.0000x reference)
import functools

import jax
import jax.numpy as jnp
from jax.experimental import pallas as pl
from jax.experimental.pallas import tpu as pltpu

INPUT_DIM = 768
NUM_PRIMES = 8192
K = 8
AMP_DIM = 4
AK = K * AMP_DIM

ROW_BLOCK = 1024
PRIME_TILE = 2048
NUM_TILES = NUM_PRIMES // PRIME_TILE
SLOT = 128


def _fused(hs_ref, sw_ref, sb_ref, aw_ref, ab_ref,
           idx_ref, amp_ref, sc_ref, cv_ref, ci_ref):
    j = pl.program_id(1)
    hs = hs_ref[...]

    @pl.when(j < NUM_TILES)
    def _matmul():
        sc_ref[j % 2] = jax.lax.dot_general(
            hs, sw_ref[...], (((1,), (1,)), ((), ())),
            preferred_element_type=jnp.float32) + sb_ref[...]

    @pl.when(j > 0)
    def _extract():
        jj = j - 1
        x = sc_ref[jj % 2]
        lane = jax.lax.broadcasted_iota(jnp.int32, (ROW_BLOCK, PRIME_TILE), 1)
        base = jj * PRIME_TILE
        vals, idxs = [], []
        for _ in range(K):
            m = jnp.max(x, axis=1, keepdims=True)
            hit = x == m
            pos = jnp.min(jnp.where(hit, lane, PRIME_TILE), axis=1,
                          keepdims=True)
            vals.append(m)
            idxs.append(pos + base)
            x = jnp.where(lane == pos, -jnp.inf, x)
        vpad = jnp.full((ROW_BLOCK, SLOT - K), -jnp.inf, dtype=jnp.float32)
        ipad = jnp.zeros((ROW_BLOCK, SLOT - K), dtype=jnp.int32)
        cv_ref[:, pl.ds(jj * SLOT, SLOT)] = jnp.concatenate(vals + [vpad],
                                                            axis=1)
        ci_ref[:, pl.ds(jj * SLOT, SLOT)] = jnp.concatenate(idxs + [ipad],
                                                            axis=1)

    @pl.when(j == NUM_TILES)
    def _merge():
        nc = NUM_TILES * SLOT
        cv = cv_ref[...]
        ci = ci_ref[...]
        slot = jax.lax.broadcasted_iota(jnp.int32, (ROW_BLOCK, nc), 1)
        x2 = cv
        fv, fi = [], []
        for _ in range(K):
            m = jnp.max(x2, axis=1, keepdims=True)
            pos = jnp.min(jnp.where(x2 == m, slot, nc), axis=1, keepdims=True)
            sel = slot == pos
            fv.append(m)
            fi.append(jnp.sum(jnp.where(sel, ci, 0), axis=1, keepdims=True))
            x2 = jnp.where(sel, -jnp.inf, x2)
        topv = jnp.concatenate(fv, axis=1)
        idx_ref[...] = jnp.concatenate(fi, axis=1)

        w = jnp.exp(topv - topv[:, :1])
        w = w / jnp.sum(w, axis=1, keepdims=True)

        amps = jax.lax.dot_general(
            hs, aw_ref[...], (((1,), (1,)), ((), ())),
            preferred_element_type=jnp.float32) + ab_ref[...]

        r8 = jax.lax.broadcasted_iota(jnp.int32, (K, AK), 0)
        c32 = jax.lax.broadcasted_iota(jnp.int32, (K, AK), 1)
        expand = (c32 // AMP_DIM == r8).astype(jnp.float32)
        w32 = jax.lax.dot_general(
            w, expand, (((1,), (0,)), ((), ())),
            preferred_element_type=jnp.float32)
        wa = amps * w32
        g = wa * wa
        p = jax.lax.broadcasted_iota(jnp.int32, (AK, AK), 0)
        q = jax.lax.broadcasted_iota(jnp.int32, (AK, AK), 1)
        gsum = (p // AMP_DIM == q // AMP_DIM).astype(jnp.float32)
        n2 = jax.lax.dot_general(
            g, gsum, (((1,), (0,)), ((), ())),
            preferred_element_type=jnp.float32)
        amp_ref[...] = wa / jnp.maximum(jnp.sqrt(n2), 1e-12)


@functools.partial(jax.jit, static_argnames=())
def kernel(hidden_states, score_w, score_b, amp_w, amp_b):
    batch, seq, d = hidden_states.shape
    rows = batch * seq
    hs2 = hidden_states.reshape(rows, d)
    sb2 = score_b.reshape(1, NUM_PRIMES)
    ab2 = amp_b.reshape(1, AK)
    nr = rows // ROW_BLOCK

    idx_out, amp_out = pl.pallas_call(
        _fused,
        grid=(nr, NUM_TILES + 1),
        in_specs=[
            pl.BlockSpec((ROW_BLOCK, d), lambda i, j: (i, 0)),
            pl.BlockSpec((PRIME_TILE, d),
                         lambda i, j: (jnp.minimum(j, NUM_TILES - 1), 0)),
            pl.BlockSpec((1, PRIME_TILE),
                         lambda i, j: (0, jnp.minimum(j, NUM_TILES - 1))),
            pl.BlockSpec((AK, d), lambda i, j: (0, 0)),
            pl.BlockSpec((1, AK), lambda i, j: (0, 0)),
        ],
        out_specs=[
            pl.BlockSpec((ROW_BLOCK, K), lambda i, j: (i, 0)),
            pl.BlockSpec((ROW_BLOCK, AK), lambda i, j: (i, 0)),
        ],
        out_shape=[
            jax.ShapeDtypeStruct((rows, K), jnp.int32),
            jax.ShapeDtypeStruct((rows, AK), jnp.float32),
        ],
        scratch_shapes=[
            pltpu.VMEM((2, ROW_BLOCK, PRIME_TILE), jnp.float32),
            pltpu.VMEM((ROW_BLOCK, NUM_TILES * SLOT), jnp.float32),
            pltpu.VMEM((ROW_BLOCK, NUM_TILES * SLOT), jnp.int32),
        ],
        compiler_params=pltpu.CompilerParams(
            dimension_semantics=("parallel", "arbitrary")),
    )(hs2, score_w, sb2, amp_w, ab2)

    topk_indices = idx_out.reshape(batch, seq, K)
    amps = amp_out.reshape(batch, seq, K, AMP_DIM)
    return (topk_indices, amps)


# speedup vs baseline: 1.3626x; 1.3626x over previous
import functools

import jax
import jax.numpy as jnp
from jax.experimental import pallas as pl
from jax.experimental.pallas import tpu as pltpu

INPUT_DIM = 768
NUM_PRIMES = 8192
K = 8
AMP_DIM = 4
AK = K * AMP_DIM

ROW_BLOCK = 1024
PRIME_TILE = 1024
NUM_TILES = NUM_PRIMES // PRIME_TILE
NCOL = PRIME_TILE // 128

_SORT8 = ((0, 1), (2, 3), (4, 5), (6, 7),
          (0, 2), (1, 3), (4, 6), (5, 7),
          (1, 2), (5, 6),
          (0, 4), (1, 5), (2, 6), (3, 7),
          (2, 4), (3, 5),
          (1, 2), (3, 4), (5, 6))

_BITONIC8 = ((0, 4), (1, 5), (2, 6), (3, 7),
             (0, 2), (1, 3), (4, 6), (5, 7),
             (0, 1), (2, 3), (4, 5), (6, 7))


def _cmpx(vs, ids, a, b):
    c = vs[a] >= vs[b]
    hv = jnp.where(c, vs[a], vs[b])
    hi = jnp.where(c, ids[a], ids[b])
    lv = jnp.where(c, vs[b], vs[a])
    li = jnp.where(c, ids[b], ids[a])
    vs[a], ids[a], vs[b], ids[b] = hv, hi, lv, li


def _sort8(vs, ids):
    vs, ids = list(vs), list(ids)
    for a, b in _SORT8:
        _cmpx(vs, ids, a, b)
    return vs, ids


def _merge_top8(av, ai, bv, bi):
    mv, mi = [], []
    for i in range(K):
        c = av[i] >= bv[K - 1 - i]
        mv.append(jnp.where(c, av[i], bv[K - 1 - i]))
        mi.append(jnp.where(c, ai[i], bi[K - 1 - i]))
    for a, b in _BITONIC8:
        _cmpx(mv, mi, a, b)
    return mv, mi


def _fused(hs_ref, sw_ref, sb_ref, aw_ref, ab_ref,
           idx_ref, amp_ref, lv_ref, li_ref):
    j = pl.program_id(1)
    hs = hs_ref[...]
    x = jax.lax.dot_general(
        hs, sw_ref[...], (((1,), (1,)), ((), ())),
        preferred_element_type=jnp.float32) + sb_ref[...]

    lane = jax.lax.broadcasted_iota(jnp.int32, (ROW_BLOCK, 128), 1)
    base = j * PRIME_TILE
    cols_v = [x[:, c * 128:(c + 1) * 128] for c in range(NCOL)]
    cols_i = [lane + (base + c * 128) for c in range(NCOL)]
    tv, ti = _sort8(cols_v, cols_i)

    @pl.when(j == 0)
    def _init():
        for k in range(K):
            lv_ref[k] = tv[k]
            li_ref[k] = ti[k]

    @pl.when(j > 0)
    def _accum():
        rv = [lv_ref[k] for k in range(K)]
        ri = [li_ref[k] for k in range(K)]
        gv, gi = _merge_top8(rv, ri, tv, ti)
        for k in range(K):
            lv_ref[k] = gv[k]
            li_ref[k] = gi[k]

    @pl.when(j == NUM_TILES - 1)
    def _finish():
        gv = [lv_ref[k] for k in range(K)]
        gi = [li_ref[k] for k in range(K)]
        fv, fi = [], []
        for _ in range(K):
            m = jnp.max(gv[0], axis=1, keepdims=True)
            pos = jnp.min(jnp.where(gv[0] == m, lane, 128), axis=1,
                          keepdims=True)
            sel = lane == pos
            fv.append(m)
            fi.append(jnp.sum(jnp.where(sel, gi[0], 0), axis=1,
                              keepdims=True))
            for k in range(K - 1):
                gv[k] = jnp.where(sel, gv[k + 1], gv[k])
                gi[k] = jnp.where(sel, gi[k + 1], gi[k])
            gv[K - 1] = jnp.where(sel, -jnp.inf, gv[K - 1])
        topv = jnp.concatenate(fv, axis=1)
        idx_ref[...] = jnp.concatenate(fi, axis=1)

        w = jnp.exp(topv - topv[:, :1])
        w = w / jnp.sum(w, axis=1, keepdims=True)

        amps = jax.lax.dot_general(
            hs, aw_ref[...], (((1,), (1,)), ((), ())),
            preferred_element_type=jnp.float32) + ab_ref[...]

        r8 = jax.lax.broadcasted_iota(jnp.int32, (K, AK), 0)
        c32 = jax.lax.broadcasted_iota(jnp.int32, (K, AK), 1)
        expand = (c32 // AMP_DIM == r8).astype(jnp.float32)
        w32 = jax.lax.dot_general(
            w, expand, (((1,), (0,)), ((), ())),
            preferred_element_type=jnp.float32)
        wa = amps * w32
        g = wa * wa
        p = jax.lax.broadcasted_iota(jnp.int32, (AK, AK), 0)
        q = jax.lax.broadcasted_iota(jnp.int32, (AK, AK), 1)
        gsum = (p // AMP_DIM == q // AMP_DIM).astype(jnp.float32)
        n2 = jax.lax.dot_general(
            g, gsum, (((1,), (0,)), ((), ())),
            preferred_element_type=jnp.float32)
        amp_ref[...] = wa / jnp.maximum(jnp.sqrt(n2), 1e-12)


@functools.partial(jax.jit, static_argnames=())
def kernel(hidden_states, score_w, score_b, amp_w, amp_b):
    batch, seq, d = hidden_states.shape
    rows = batch * seq
    hs2 = hidden_states.reshape(rows, d)
    sb2 = score_b.reshape(1, NUM_PRIMES)
    ab2 = amp_b.reshape(1, AK)
    nr = rows // ROW_BLOCK

    idx_out, amp_out = pl.pallas_call(
        _fused,
        grid=(nr, NUM_TILES),
        in_specs=[
            pl.BlockSpec((ROW_BLOCK, d), lambda i, j: (i, 0)),
            pl.BlockSpec((PRIME_TILE, d), lambda i, j: (j, 0)),
            pl.BlockSpec((1, PRIME_TILE), lambda i, j: (0, j)),
            pl.BlockSpec((AK, d), lambda i, j: (0, 0)),
            pl.BlockSpec((1, AK), lambda i, j: (0, 0)),
        ],
        out_specs=[
            pl.BlockSpec((ROW_BLOCK, K), lambda i, j: (i, 0)),
            pl.BlockSpec((ROW_BLOCK, AK), lambda i, j: (i, 0)),
        ],
        out_shape=[
            jax.ShapeDtypeStruct((rows, K), jnp.int32),
            jax.ShapeDtypeStruct((rows, AK), jnp.float32),
        ],
        scratch_shapes=[
            pltpu.VMEM((K, ROW_BLOCK, 128), jnp.float32),
            pltpu.VMEM((K, ROW_BLOCK, 128), jnp.int32),
        ],
        compiler_params=pltpu.CompilerParams(
            dimension_semantics=("parallel", "arbitrary")),
    )(hs2, score_w, sb2, amp_w, ab2)

    topk_indices = idx_out.reshape(batch, seq, K)
    amps = amp_out.reshape(batch, seq, K, AMP_DIM)
    return (topk_indices, amps)
